# baseline (speedup 1.0000x reference)
import functools

import jax
import jax.numpy as jnp
from jax import lax
from jax.experimental import pallas as pl
from jax.experimental.pallas import tpu as pltpu
from jax.experimental.pallas import tpu_sc as plsc

_L = 16
_CHAINS = 2
_NUM_CORES = 2


def _row_argmax(w_v, lane, N):
  nsteps = N // _L
  def sweep(o, carry):
    accs = list(carry)
    base = o * (2 * _CHAINS)
    for u in range(2 * _CHAINS):
      a = u % _CHAINS
      j = base + u
      bv, bj = accs[a]
      vals = w_v[pl.ds(j * _L, _L)]
      m = vals > bv
      bv = jnp.where(m, vals, bv)
      bj = jnp.where(m, j, bj)
      accs[a] = (bv, bj)
    return tuple(accs)

  init = tuple(
      (jnp.full((_L,), -jnp.inf, jnp.float32), jnp.zeros((_L,), jnp.int32))
      for _ in range(_CHAINS))
  accs = lax.fori_loop(0, nsteps // (2 * _CHAINS), sweep, init)
  bv, bj = accs[0]
  for ov, oj in accs[1:]:
    better = (ov > bv) | ((ov == bv) & (oj < bj))
    bv = jnp.where(better, ov, bv)
    bj = jnp.where(better, oj, bj)
  bi = bj * _L + lane
  for k in (8, 4, 2, 1):
    perm = lane ^ k
    ov = bv.at[perm].get(mode="promise_in_bounds")
    oi = bi.at[perm].get(mode="promise_in_bounds")
    better = (ov > bv) | ((ov == bv) & (oi < bi))
    bv = jnp.where(better, ov, bv)
    bi = jnp.where(better, oi, bi)
  return bv[0], bi[0]


def _scale_row(row_v, scale, D):
  def body(o, _):
    for u in range(4):
      idx = pl.ds((o * 4 + u) * _L, _L)
      row_v[idx] = row_v[idx] * scale
    return 0

  lax.fori_loop(0, D // (4 * _L), body, 0)


def _select_body(N, D, R, sw_hbm, mem_hbm, out_hbm, *scratch):
  w_refs = scratch[:R]
  row_refs = scratch[R:2 * R]
  sems = scratch[2 * R:3 * R]

  sid = lax.axis_index("s")
  if _NUM_CORES > 1:
    wid = sid * _NUM_CORES + lax.axis_index("c")
  else:
    wid = sid
  lane = lax.iota(jnp.int32, _L)
  one = jnp.float32(1.0)

  bs = [wid * R + r for r in range(R)]

  w_cps = [pltpu.async_copy(sw_hbm.at[bs[r]], w_refs[r], sems[r])
           for r in range(R)]
  r_cps, scales = [], []
  for r in range(R):
    w_cps[r].wait()
    v, i = _row_argmax(w_refs[r], lane, N)
    scales.append(v + (one - v))
    r_cps.append(
        pltpu.async_copy(mem_hbm.at[bs[r] * N + i], row_refs[r], sems[r]))
  o_cps = []
  for r in range(R):
    r_cps[r].wait()
    sc = scales[r]
    pl.when(sc != one)(functools.partial(_scale_row, row_refs[r], sc, D))
    o_cps.append(pltpu.async_copy(row_refs[r], out_hbm.at[bs[r]], sems[r]))
  for cp in o_cps:
    cp.wait()


def kernel(previous_encoded_m, sim_weights):
  B, N = sim_weights.shape
  D = previous_encoded_m.shape[2]
  R = B // (16 * _NUM_CORES)

  table = previous_encoded_m.reshape(B * N, D)

  mesh = plsc.VectorSubcoreMesh(
      core_axis_name="c", subcore_axis_name="s", num_cores=_NUM_CORES)
  run = pl.kernel(
      functools.partial(_select_body, N, D, R),
      mesh=mesh,
      out_type=jax.ShapeDtypeStruct((B, D), jnp.float32),
      scratch_types=(
          [pltpu.VMEM((N,), jnp.float32)] * R
          + [pltpu.VMEM((D,), jnp.float32)] * R
          + [pltpu.SemaphoreType.DMA] * R
      ),
  )
  return run(sim_weights, table)


# speedup vs baseline: 1.0652x; 1.0652x over previous
import functools

import jax
import jax.numpy as jnp
from jax import lax
from jax.experimental import pallas as pl
from jax.experimental.pallas import tpu as pltpu
from jax.experimental.pallas import tpu_sc as plsc

_L = 16
_CHAINS = 2
_NUM_CORES = 1


def _row_argmax(w_v, lane, N):
  nsteps = N // _L
  def sweep(o, carry):
    accs = list(carry)
    base = o * (2 * _CHAINS)
    for u in range(2 * _CHAINS):
      a = u % _CHAINS
      j = base + u
      bv, bj = accs[a]
      vals = w_v[pl.ds(j * _L, _L)]
      m = vals > bv
      bv = jnp.where(m, vals, bv)
      bj = jnp.where(m, j, bj)
      accs[a] = (bv, bj)
    return tuple(accs)

  init = tuple(
      (jnp.full((_L,), -jnp.inf, jnp.float32), jnp.zeros((_L,), jnp.int32))
      for _ in range(_CHAINS))
  accs = lax.fori_loop(0, nsteps // (2 * _CHAINS), sweep, init)
  bv, bj = accs[0]
  for ov, oj in accs[1:]:
    better = (ov > bv) | ((ov == bv) & (oj < bj))
    bv = jnp.where(better, ov, bv)
    bj = jnp.where(better, oj, bj)
  bi = bj * _L + lane
  for k in (8, 4, 2, 1):
    perm = lane ^ k
    ov = bv.at[perm].get(mode="promise_in_bounds")
    oi = bi.at[perm].get(mode="promise_in_bounds")
    better = (ov > bv) | ((ov == bv) & (oi < bi))
    bv = jnp.where(better, ov, bv)
    bi = jnp.where(better, oi, bi)
  return bv[0], bi[0]


def _scale_row(row_v, scale, D):
  def body(o, _):
    for u in range(4):
      idx = pl.ds((o * 4 + u) * _L, _L)
      row_v[idx] = row_v[idx] * scale
    return 0

  lax.fori_loop(0, D // (4 * _L), body, 0)


def _select_body(N, D, R, sw_hbm, mem_hbm, out_hbm, *scratch):
  w_refs = scratch[:R]
  row_refs = scratch[R:2 * R]
  sems = scratch[2 * R:3 * R]

  sid = lax.axis_index("s")
  if _NUM_CORES > 1:
    wid = sid * _NUM_CORES + lax.axis_index("c")
  else:
    wid = sid
  lane = lax.iota(jnp.int32, _L)
  one = jnp.float32(1.0)

  bs = [wid * R + r for r in range(R)]

  w_cps = [pltpu.async_copy(sw_hbm.at[bs[r]], w_refs[r], sems[r])
           for r in range(R)]
  r_cps, scales = [], []
  for r in range(R):
    w_cps[r].wait()
    v, i = _row_argmax(w_refs[r], lane, N)
    scales.append(v + (one - v))
    r_cps.append(
        pltpu.async_copy(mem_hbm.at[bs[r] * N + i], row_refs[r], sems[r]))
  o_cps = []
  for r in range(R):
    r_cps[r].wait()
    sc = scales[r]
    pl.when(sc != one)(functools.partial(_scale_row, row_refs[r], sc, D))
    o_cps.append(pltpu.async_copy(row_refs[r], out_hbm.at[bs[r]], sems[r]))
  for cp in o_cps:
    cp.wait()


def kernel(previous_encoded_m, sim_weights):
  B, N = sim_weights.shape
  D = previous_encoded_m.shape[2]
  R = B // (16 * _NUM_CORES)

  table = previous_encoded_m.reshape(B * N, D)

  mesh = plsc.VectorSubcoreMesh(
      core_axis_name="c", subcore_axis_name="s", num_cores=_NUM_CORES)
  run = pl.kernel(
      functools.partial(_select_body, N, D, R),
      mesh=mesh,
      out_type=jax.ShapeDtypeStruct((B, D), jnp.float32),
      scratch_types=(
          [pltpu.VMEM((N,), jnp.float32)] * R
          + [pltpu.VMEM((D,), jnp.float32)] * R
          + [pltpu.SemaphoreType.DMA] * R
      ),
  )
  return run(sim_weights, table)


# speedup vs baseline: 1.0718x; 1.0062x over previous
import functools

import jax
import jax.numpy as jnp
from jax import lax
from jax.experimental import pallas as pl
from jax.experimental.pallas import tpu as pltpu
from jax.experimental.pallas import tpu_sc as plsc

_L = 16
_CHAINS = 4
_NUM_CORES = 1


def _row_argmax(w_v, lane, N):
  nsteps = N // _L
  def sweep(o, carry):
    accs = list(carry)
    base = o * (2 * _CHAINS)
    for u in range(2 * _CHAINS):
      a = u % _CHAINS
      j = base + u
      bv, bj = accs[a]
      vals = w_v[pl.ds(j * _L, _L)]
      m = vals > bv
      bv = jnp.where(m, vals, bv)
      bj = jnp.where(m, j, bj)
      accs[a] = (bv, bj)
    return tuple(accs)

  init = tuple(
      (jnp.full((_L,), -jnp.inf, jnp.float32), jnp.zeros((_L,), jnp.int32))
      for _ in range(_CHAINS))
  accs = lax.fori_loop(0, nsteps // (2 * _CHAINS), sweep, init)
  bv, bj = accs[0]
  for ov, oj in accs[1:]:
    better = (ov > bv) | ((ov == bv) & (oj < bj))
    bv = jnp.where(better, ov, bv)
    bj = jnp.where(better, oj, bj)
  bi = bj * _L + lane
  for k in (8, 4, 2, 1):
    perm = lane ^ k
    ov = bv.at[perm].get(mode="promise_in_bounds")
    oi = bi.at[perm].get(mode="promise_in_bounds")
    better = (ov > bv) | ((ov == bv) & (oi < bi))
    bv = jnp.where(better, ov, bv)
    bi = jnp.where(better, oi, bi)
  return bv[0], bi[0]


def _scale_row(row_v, scale, D):
  def body(o, _):
    for u in range(4):
      idx = pl.ds((o * 4 + u) * _L, _L)
      row_v[idx] = row_v[idx] * scale
    return 0

  lax.fori_loop(0, D // (4 * _L), body, 0)


def _select_body(N, D, R, sw_hbm, mem_hbm, out_hbm, *scratch):
  w_refs = scratch[:R]
  row_refs = scratch[R:2 * R]
  sems = scratch[2 * R:3 * R]

  sid = lax.axis_index("s")
  if _NUM_CORES > 1:
    wid = sid * _NUM_CORES + lax.axis_index("c")
  else:
    wid = sid
  lane = lax.iota(jnp.int32, _L)
  one = jnp.float32(1.0)

  bs = [wid * R + r for r in range(R)]

  w_cps = [pltpu.async_copy(sw_hbm.at[bs[r]], w_refs[r], sems[r])
           for r in range(R)]
  r_cps, scales = [], []
  for r in range(R):
    w_cps[r].wait()
    v, i = _row_argmax(w_refs[r], lane, N)
    scales.append(v + (one - v))
    r_cps.append(
        pltpu.async_copy(mem_hbm.at[bs[r] * N + i], row_refs[r], sems[r]))
  o_cps = []
  for r in range(R):
    r_cps[r].wait()
    sc = scales[r]
    pl.when(sc != one)(functools.partial(_scale_row, row_refs[r], sc, D))
    o_cps.append(pltpu.async_copy(row_refs[r], out_hbm.at[bs[r]], sems[r]))
  for cp in o_cps:
    cp.wait()


def kernel(previous_encoded_m, sim_weights):
  B, N = sim_weights.shape
  D = previous_encoded_m.shape[2]
  R = B // (16 * _NUM_CORES)

  table = previous_encoded_m.reshape(B * N, D)

  mesh = plsc.VectorSubcoreMesh(
      core_axis_name="c", subcore_axis_name="s", num_cores=_NUM_CORES)
  run = pl.kernel(
      functools.partial(_select_body, N, D, R),
      mesh=mesh,
      out_type=jax.ShapeDtypeStruct((B, D), jnp.float32),
      scratch_types=(
          [pltpu.VMEM((N,), jnp.float32)] * R
          + [pltpu.VMEM((D,), jnp.float32)] * R
          + [pltpu.SemaphoreType.DMA] * R
      ),
  )
  return run(sim_weights, table)
